# baseline (speedup 1.0000x reference)
import functools

import jax
import jax.numpy as jnp
from jax import lax
from jax.experimental import pallas as pl
from jax.experimental.pallas import tpu as pltpu
from jax.experimental.pallas import tpu_sc as plsc

FDIM = 32
G = 129
V = G * G * G

NC = 2
NS = 16
NW = NC * NS
L = 16

BLK = 128


def _sc_body(nblocks, xc, yc, zc, vol, out, cx_v, cy_v, cz_v, idx_v, w_v,
             corners_v, out_v, gsem, osem):
    wid = lax.axis_index("s") * NC + lax.axis_index("c")
    gmax = jnp.float32(G - 1)

    def block_body(i, carry):
        base = (wid * nblocks + i) * BLK
        pltpu.sync_copy(xc.at[pl.ds(base, BLK)], cx_v)
        pltpu.sync_copy(yc.at[pl.ds(base, BLK)], cy_v)
        pltpu.sync_copy(zc.at[pl.ds(base, BLK)], cz_v)

        for g in range(BLK // L):
            sl = pl.ds(g * L, L)
            ix = jnp.clip((cx_v[sl] + 1.0) * (0.5 * (G - 1)), 0.0, gmax)
            iy = jnp.clip((cy_v[sl] + 1.0) * (0.5 * (G - 1)), 0.0, gmax)
            iz = jnp.clip((cz_v[sl] + 1.0) * (0.5 * (G - 1)), 0.0, gmax)
            x0 = ix.astype(jnp.int32)
            y0 = iy.astype(jnp.int32)
            z0 = iz.astype(jnp.int32)
            fx = ix - x0.astype(jnp.float32)
            fy = iy - y0.astype(jnp.float32)
            fz = iz - z0.astype(jnp.float32)
            dx = jnp.minimum(x0 + 1, G - 1) - x0
            dy = (jnp.minimum(y0 + 1, G - 1) - y0) * G
            dz = (jnp.minimum(z0 + 1, G - 1) - z0) * (G * G)
            base000 = z0 * (G * G) + y0 * G + x0
            gx = (1.0 - fx, fx)
            gy = (1.0 - fy, fy)
            gz = (1.0 - fz, fz)
            ox = (jnp.zeros((L,), jnp.int32), dx)
            oy = (jnp.zeros((L,), jnp.int32), dy)
            oz = (jnp.zeros((L,), jnp.int32), dz)
            k = 0
            for kz in range(2):
                for ky in range(2):
                    for kx in range(2):
                        idx_v[k, sl] = base000 + oz[kz] + oy[ky] + ox[kx]
                        w_v[k, sl] = gz[kz] * gy[ky] * gx[kx]
                        k += 1

        for k in range(8):
            pltpu.async_copy(vol.at[idx_v.at[k]], corners_v.at[k], gsem)
        for k in range(8):
            pltpu.make_async_copy(vol.at[idx_v.at[0]], corners_v.at[0],
                                  gsem).wait()

        def group_body(g, carry):
            sl = pl.ds(g * L, L)
            w_rows = [w_v[k, sl] for k in range(8)]
            for q in range(L):
                p = g * L + q
                acc0 = jnp.zeros((L,), jnp.float32)
                acc1 = jnp.zeros((L,), jnp.float32)
                for k in range(8):
                    wk = w_rows[k][q]
                    acc0 = acc0 + wk * corners_v[k, p, pl.ds(0, L)]
                    acc1 = acc1 + wk * corners_v[k, p, pl.ds(L, L)]
                out_v[p, pl.ds(0, L)] = acc0
                out_v[p, pl.ds(L, L)] = acc1
            return carry

        lax.fori_loop(0, BLK // L, group_body, 0)

        copy = pltpu.make_async_copy(out_v, out.at[pl.ds(base, BLK)], osem)
        copy.start()
        copy.wait()
        return carry

    lax.fori_loop(0, nblocks, block_body, 0)


def kernel(x, fm):
    n = x.shape[0]
    nblocks = -(-n // (BLK * NW))
    npad = nblocks * NW * BLK

    vol = jnp.transpose(fm[0], (1, 2, 3, 0)).reshape(V, FDIM)
    xt = jnp.pad(x, ((0, npad - n), (0, 0))).T
    xc, yc, zc = xt[0], xt[1], xt[2]

    mesh = plsc.VectorSubcoreMesh(core_axis_name="c", subcore_axis_name="s")
    run = functools.partial(
        pl.kernel,
        mesh=mesh,
        compiler_params=pltpu.CompilerParams(use_tc_tiling_on_sc=False),
        out_type=jax.ShapeDtypeStruct((npad, FDIM), jnp.float32),
        scratch_types=[
            pltpu.VMEM((BLK,), jnp.float32),
            pltpu.VMEM((BLK,), jnp.float32),
            pltpu.VMEM((BLK,), jnp.float32),
            pltpu.VMEM((8, BLK), jnp.int32),
            pltpu.VMEM((8, BLK), jnp.float32),
            pltpu.VMEM((8, BLK, FDIM), jnp.float32),
            pltpu.VMEM((BLK, FDIM), jnp.float32),
            pltpu.SemaphoreType.DMA,
            pltpu.SemaphoreType.DMA,
        ],
    )(functools.partial(_sc_body, nblocks))
    out = run(xc, yc, zc, vol)
    return out[:n]


# speedup vs baseline: 1.5762x; 1.5762x over previous
import functools

import jax
import jax.numpy as jnp
from jax import lax
from jax.experimental import pallas as pl
from jax.experimental.pallas import tpu as pltpu
from jax.experimental.pallas import tpu_sc as plsc

FDIM = 32
G = 129
GG = G * G
V = G * G * G

NC = 2
NS = 16
NW = NC * NS
L = 16

BLK = 128

PPC = 8
NFULL_CHUNK = GG // PPC
CHUNK_VOX = PPC * G


def _relayout_body(fmv, out, inbuf, outbuf):
    wid = lax.axis_index("s") * NC + lax.axis_index("c")
    nchunk = NFULL_CHUNK + 1
    kmax = -(-nchunk // NW)
    ci0 = lax.iota(jnp.int32, L)
    ci1 = ci0 + L

    def transpose_planes(nplanes):
        def p_body(j, carry2):
            jv = jnp.full((L,), j, jnp.int32)

            def x_body(x, carry3):
                xv = jnp.full((L,), x, jnp.int32)
                v0 = plsc.load_gather(inbuf, [jv, ci0, xv])
                v1 = plsc.load_gather(inbuf, [jv, ci1, xv])
                o = (j * G + x) * FDIM
                outbuf[pl.ds(o, L)] = v0
                outbuf[pl.ds(o + L, L)] = v1
                return carry3

            return lax.fori_loop(0, G, x_body, carry2)

        lax.fori_loop(0, nplanes, p_body, 0)

    def chunk_body(k, carry):
        t = k * NW + wid

        @pl.when(t < NFULL_CHUNK)
        def _full():
            p0 = t * PPC
            for j in range(PPC):
                pltpu.sync_copy(fmv.at[p0 + j], inbuf.at[j])
            transpose_planes(PPC)
            pltpu.sync_copy(outbuf,
                            out.at[pl.ds(p0 * G * FDIM, CHUNK_VOX * FDIM)])

        @pl.when(t == NFULL_CHUNK)
        def _tail():
            p0 = NFULL_CHUNK * PPC
            pltpu.sync_copy(fmv.at[p0], inbuf.at[0])
            transpose_planes(1)
            pltpu.sync_copy(outbuf.at[pl.ds(0, G * FDIM)],
                            out.at[pl.ds(p0 * G * FDIM, G * FDIM)])

        return carry

    lax.fori_loop(0, kmax, chunk_body, 0)


def _lerp_groups(cx_v, cy_v, cz_v, idx_v, w_v, ngroups):
    gmax = jnp.float32(G - 1)
    for g in range(ngroups):
        sl = pl.ds(g * L, L)
        ix = jnp.clip((cx_v[sl] + 1.0) * (0.5 * (G - 1)), 0.0, gmax)
        iy = jnp.clip((cy_v[sl] + 1.0) * (0.5 * (G - 1)), 0.0, gmax)
        iz = jnp.clip((cz_v[sl] + 1.0) * (0.5 * (G - 1)), 0.0, gmax)
        x0 = ix.astype(jnp.int32)
        y0 = iy.astype(jnp.int32)
        z0 = iz.astype(jnp.int32)
        fx = ix - x0.astype(jnp.float32)
        fy = iy - y0.astype(jnp.float32)
        fz = iz - z0.astype(jnp.float32)
        dx = jnp.minimum(x0 + 1, G - 1) - x0
        dy = (jnp.minimum(y0 + 1, G - 1) - y0) * G
        dz = (jnp.minimum(z0 + 1, G - 1) - z0) * GG
        base000 = z0 * GG + y0 * G + x0
        gx = (1.0 - fx, fx)
        gy = (1.0 - fy, fy)
        gz = (1.0 - fz, fz)
        ox = (jnp.zeros((L,), jnp.int32), dx)
        oy = (jnp.zeros((L,), jnp.int32), dy)
        oz = (jnp.zeros((L,), jnp.int32), dz)
        k = 0
        for kz in range(2):
            for ky in range(2):
                for kx in range(2):
                    idx_v[k, sl] = base000 + oz[kz] + oy[ky] + ox[kx]
                    w_v[k, sl] = gz[kz] * gy[ky] * gx[kx]
                    k += 1


def _combine_group(g, w_v, corners_v, out_v):
    sl = pl.ds(g * L, L)
    w_rows = [w_v[k, sl] for k in range(8)]
    pv = lax.iota(jnp.int32, L) + g * L
    for c in range(FDIM):
        cv = jnp.full((L,), c, jnp.int32)
        acc = jnp.zeros((L,), jnp.float32)
        for k in range(8):
            kv = jnp.full((L,), k, jnp.int32)
            acc = acc + w_rows[k] * plsc.load_gather(corners_v, [kv, pv, cv])
        plsc.store_scatter(out_v, [pv, cv], acc)


def _lookup_body(nfull, ntail, xc, yc, zc, tbl, out, cx_v, cy_v, cz_v, idx_v,
                 w_v, corners_v, out_v, gsem, osem):
    wid = lax.axis_index("s") * NC + lax.axis_index("c")
    nblocks = nfull + (1 if ntail else 0)
    kmax = -(-nblocks // NW)

    def block_body(i, carry):
        b = i * NW + wid

        @pl.when(b < nfull)
        def _full():
            base = b * BLK
            pltpu.sync_copy(xc.at[pl.ds(base, BLK)], cx_v)
            pltpu.sync_copy(yc.at[pl.ds(base, BLK)], cy_v)
            pltpu.sync_copy(zc.at[pl.ds(base, BLK)], cz_v)
            _lerp_groups(cx_v, cy_v, cz_v, idx_v, w_v, BLK // L)
            for k in range(8):
                pltpu.async_copy(tbl.at[idx_v.at[k]], corners_v.at[k], gsem)
            for k in range(8):
                pltpu.make_async_copy(tbl.at[idx_v.at[0]], corners_v.at[0],
                                      gsem).wait()

            def group_body(g, carry2):
                _combine_group(g, w_v, corners_v, out_v)
                return carry2

            lax.fori_loop(0, BLK // L, group_body, 0)
            copy = pltpu.make_async_copy(out_v, out.at[pl.ds(base, BLK)],
                                         osem)
            copy.start()
            copy.wait()

        if ntail:
            @pl.when(b == nfull)
            def _tail():
                base = nfull * BLK
                pltpu.sync_copy(xc.at[pl.ds(base, ntail)],
                                cx_v.at[pl.ds(0, ntail)])
                pltpu.sync_copy(yc.at[pl.ds(base, ntail)],
                                cy_v.at[pl.ds(0, ntail)])
                pltpu.sync_copy(zc.at[pl.ds(base, ntail)],
                                cz_v.at[pl.ds(0, ntail)])
                _lerp_groups(cx_v, cy_v, cz_v, idx_v, w_v, ntail // L)
                for k in range(8):
                    pltpu.async_copy(tbl.at[idx_v.at[k, pl.ds(0, ntail)]],
                                     corners_v.at[k, pl.ds(0, ntail), :],
                                     gsem)
                for k in range(8):
                    pltpu.make_async_copy(
                        tbl.at[idx_v.at[0, pl.ds(0, ntail)]],
                        corners_v.at[0, pl.ds(0, ntail), :], gsem).wait()

                def group_body(g, carry2):
                    _combine_group(g, w_v, corners_v, out_v)
                    return carry2

                lax.fori_loop(0, ntail // L, group_body, 0)
                copy = pltpu.make_async_copy(
                    out_v.at[pl.ds(0, ntail), :],
                    out.at[pl.ds(base, ntail)], osem)
                copy.start()
                copy.wait()

        return carry

    lax.fori_loop(0, kmax, block_body, 0)


def kernel(x, fm):
    n = x.shape[0]
    nfull = n // BLK
    ntail = n - nfull * BLK

    mesh = plsc.VectorSubcoreMesh(core_axis_name="c", subcore_axis_name="s")

    fmv = jnp.transpose(fm[0], (1, 2, 0, 3)).reshape(GG, FDIM, G)
    relayout = functools.partial(
        pl.kernel,
        mesh=mesh,
        compiler_params=pltpu.CompilerParams(use_tc_tiling_on_sc=True,
                                             needs_layout_passes=False),
        out_type=jax.ShapeDtypeStruct((V * FDIM,), jnp.float32),
        scratch_types=[
            pltpu.VMEM((PPC, FDIM, G), jnp.float32),
            pltpu.VMEM((CHUNK_VOX * FDIM,), jnp.float32),
        ],
    )(_relayout_body)
    tbl = relayout(fmv).reshape(V, FDIM)

    xt = x.T
    lookup = functools.partial(
        pl.kernel,
        mesh=mesh,
        compiler_params=pltpu.CompilerParams(use_tc_tiling_on_sc=False,
                                             needs_layout_passes=False),
        out_type=jax.ShapeDtypeStruct((n, FDIM), jnp.float32),
        scratch_types=[
            pltpu.VMEM((BLK,), jnp.float32),
            pltpu.VMEM((BLK,), jnp.float32),
            pltpu.VMEM((BLK,), jnp.float32),
            pltpu.VMEM((8, BLK), jnp.int32),
            pltpu.VMEM((8, BLK), jnp.float32),
            pltpu.VMEM((8, BLK, FDIM), jnp.float32),
            pltpu.VMEM((BLK, FDIM), jnp.float32),
            pltpu.SemaphoreType.DMA,
            pltpu.SemaphoreType.DMA,
        ],
    )(functools.partial(_lookup_body, nfull, ntail))
    return lookup(xt[0], xt[1], xt[2], tbl)


# speedup vs baseline: 4.1723x; 2.6470x over previous
import functools

import jax
import jax.numpy as jnp
from jax import lax
from jax.experimental import pallas as pl
from jax.experimental.pallas import tpu as pltpu
from jax.experimental.pallas import tpu_sc as plsc

FDIM = 32
G = 129
GG = G * G
V = G * G * G

NC = 2
NS = 16
NW = NC * NS
L = 16

BLK = 128

PPC = 8
NFULL_CHUNK = GG // PPC
CHUNK_VOX = PPC * G


def _relayout_body(fmv, out, inbuf, outbuf, isem, osem):
    wid = lax.axis_index("s") * NC + lax.axis_index("c")
    nchunk = NFULL_CHUNK + 1
    kmax = -(-nchunk // NW)
    xi = lax.iota(jnp.int32, L)
    xstarts = [xg * L for xg in range(8)] + [G - L]
    bxv = [(xi + s) * FDIM for s in xstarts]

    def do_plane(j, p0):
        joff = j * G * FDIM
        jv = jnp.full((L,), j, jnp.int32)
        xv_tail = xi + (G - L)

        def c_body(c, carry2):
            off = joff + c
            for xg in range(8):
                v = inbuf[j, c, pl.ds(xstarts[xg], L)]
                plsc.store_scatter(outbuf, [bxv[xg] + off], v)
            cv = jnp.full((L,), c, jnp.int32)
            v = plsc.load_gather(inbuf, [jv, cv, xv_tail])
            plsc.store_scatter(outbuf, [bxv[8] + off], v)
            return carry2

        lax.fori_loop(0, FDIM, c_body, 0)
        copy = pltpu.make_async_copy(
            outbuf.at[pl.ds(joff, G * FDIM)],
            out.at[pl.ds((p0 + j) * G * FDIM, G * FDIM)], osem)
        copy.start()
        return copy

    def chunk_body(k, carry):
        t = k * NW + wid

        @pl.when(t < NFULL_CHUNK)
        def _full():
            p0 = t * PPC
            incopies = []
            for j in range(PPC):
                incopies.append(pltpu.make_async_copy(
                    fmv.at[p0 + j], inbuf.at[j], isem))
                incopies[-1].start()
            outcopies = []
            for j in range(PPC):
                incopies[j].wait()
                outcopies.append(do_plane(j, p0))
            for c in outcopies:
                c.wait()

        @pl.when(t == NFULL_CHUNK)
        def _tail():
            p0 = NFULL_CHUNK * PPC
            copy = pltpu.make_async_copy(fmv.at[p0], inbuf.at[0], isem)
            copy.start()
            copy.wait()
            do_plane(0, p0).wait()

        return carry

    lax.fori_loop(0, kmax, chunk_body, 0)


def _lerp_groups(cx_v, cy_v, cz_v, idx_v, w_v, ngroups):
    gmax = jnp.float32(G - 1)
    for g in range(ngroups):
        sl = pl.ds(g * L, L)
        ix = jnp.clip((cx_v[sl] + 1.0) * (0.5 * (G - 1)), 0.0, gmax)
        iy = jnp.clip((cy_v[sl] + 1.0) * (0.5 * (G - 1)), 0.0, gmax)
        iz = jnp.clip((cz_v[sl] + 1.0) * (0.5 * (G - 1)), 0.0, gmax)
        x0 = ix.astype(jnp.int32)
        y0 = iy.astype(jnp.int32)
        z0 = iz.astype(jnp.int32)
        fx = ix - x0.astype(jnp.float32)
        fy = iy - y0.astype(jnp.float32)
        fz = iz - z0.astype(jnp.float32)
        dx = jnp.minimum(x0 + 1, G - 1) - x0
        dy = (jnp.minimum(y0 + 1, G - 1) - y0) * G
        dz = (jnp.minimum(z0 + 1, G - 1) - z0) * GG
        base000 = z0 * GG + y0 * G + x0
        gx = (1.0 - fx, fx)
        gy = (1.0 - fy, fy)
        gz = (1.0 - fz, fz)
        ox = (jnp.zeros((L,), jnp.int32), dx)
        oy = (jnp.zeros((L,), jnp.int32), dy)
        oz = (jnp.zeros((L,), jnp.int32), dz)
        k = 0
        for kz in range(2):
            for ky in range(2):
                for kx in range(2):
                    idx_v[k, sl] = base000 + oz[kz] + oy[ky] + ox[kx]
                    w_v[k, sl] = gz[kz] * gy[ky] * gx[kx]
                    k += 1


def _combine_group(g, w_v, corners_v, out_v):
    sl = pl.ds(g * L, L)
    w_rows = [w_v[k, sl] for k in range(8)]
    for q in range(L):
        p = g * L + q
        acc0 = jnp.zeros((L,), jnp.float32)
        acc1 = jnp.zeros((L,), jnp.float32)
        for k in range(8):
            wk = w_rows[k][q]
            acc0 = acc0 + wk * corners_v[k, p, pl.ds(0, L)]
            acc1 = acc1 + wk * corners_v[k, p, pl.ds(L, L)]
        out_v[p, pl.ds(0, L)] = acc0
        out_v[p, pl.ds(L, L)] = acc1


def _lookup_body(nfull, ntail, xc, yc, zc, tbl, out, cx_v, cy_v, cz_v, idx_v,
                 w_v, corners_v, out_v, gsem, osem):
    wid = lax.axis_index("s") * NC + lax.axis_index("c")
    nblocks = nfull + (1 if ntail else 0)
    kmax = -(-nblocks // NW)

    def block_body(i, carry):
        b = i * NW + wid

        @pl.when(b < nfull)
        def _full():
            base = b * BLK
            pltpu.sync_copy(xc.at[pl.ds(base, BLK)], cx_v)
            pltpu.sync_copy(yc.at[pl.ds(base, BLK)], cy_v)
            pltpu.sync_copy(zc.at[pl.ds(base, BLK)], cz_v)
            _lerp_groups(cx_v, cy_v, cz_v, idx_v, w_v, BLK // L)
            for k in range(8):
                pltpu.async_copy(tbl.at[idx_v.at[k]], corners_v.at[k], gsem)
            for k in range(8):
                pltpu.make_async_copy(tbl.at[idx_v.at[0]], corners_v.at[0],
                                      gsem).wait()

            def group_body(g, carry2):
                _combine_group(g, w_v, corners_v, out_v)
                return carry2

            lax.fori_loop(0, BLK // L, group_body, 0)
            copy = pltpu.make_async_copy(out_v, out.at[pl.ds(base, BLK)],
                                         osem)
            copy.start()
            copy.wait()

        if ntail:
            @pl.when(b == nfull)
            def _tail():
                base = nfull * BLK
                pltpu.sync_copy(xc.at[pl.ds(base, ntail)],
                                cx_v.at[pl.ds(0, ntail)])
                pltpu.sync_copy(yc.at[pl.ds(base, ntail)],
                                cy_v.at[pl.ds(0, ntail)])
                pltpu.sync_copy(zc.at[pl.ds(base, ntail)],
                                cz_v.at[pl.ds(0, ntail)])
                _lerp_groups(cx_v, cy_v, cz_v, idx_v, w_v, ntail // L)
                for k in range(8):
                    pltpu.async_copy(tbl.at[idx_v.at[k, pl.ds(0, ntail)]],
                                     corners_v.at[k, pl.ds(0, ntail), :],
                                     gsem)
                for k in range(8):
                    pltpu.make_async_copy(
                        tbl.at[idx_v.at[0, pl.ds(0, ntail)]],
                        corners_v.at[0, pl.ds(0, ntail), :], gsem).wait()

                def group_body(g, carry2):
                    _combine_group(g, w_v, corners_v, out_v)
                    return carry2

                lax.fori_loop(0, ntail // L, group_body, 0)
                copy = pltpu.make_async_copy(
                    out_v.at[pl.ds(0, ntail), :],
                    out.at[pl.ds(base, ntail)], osem)
                copy.start()
                copy.wait()

        return carry

    lax.fori_loop(0, kmax, block_body, 0)


def kernel(x, fm):
    n = x.shape[0]
    nfull = n // BLK
    ntail = n - nfull * BLK

    mesh = plsc.VectorSubcoreMesh(core_axis_name="c", subcore_axis_name="s")

    fmv = jnp.transpose(fm[0], (1, 2, 0, 3)).reshape(GG, FDIM, G)
    relayout = functools.partial(
        pl.kernel,
        mesh=mesh,
        compiler_params=pltpu.CompilerParams(use_tc_tiling_on_sc=True,
                                             needs_layout_passes=False),
        out_type=jax.ShapeDtypeStruct((V * FDIM,), jnp.float32),
        scratch_types=[
            pltpu.VMEM((PPC, FDIM, G), jnp.float32),
            pltpu.VMEM((CHUNK_VOX * FDIM,), jnp.float32),
            pltpu.SemaphoreType.DMA,
            pltpu.SemaphoreType.DMA,
        ],
    )(_relayout_body)
    tbl = relayout(fmv).reshape(V, FDIM)

    xt = x.T
    lookup = functools.partial(
        pl.kernel,
        mesh=mesh,
        compiler_params=pltpu.CompilerParams(use_tc_tiling_on_sc=False,
                                             needs_layout_passes=False),
        out_type=jax.ShapeDtypeStruct((n, FDIM), jnp.float32),
        scratch_types=[
            pltpu.VMEM((BLK,), jnp.float32),
            pltpu.VMEM((BLK,), jnp.float32),
            pltpu.VMEM((BLK,), jnp.float32),
            pltpu.VMEM((8, BLK), jnp.int32),
            pltpu.VMEM((8, BLK), jnp.float32),
            pltpu.VMEM((8, BLK, FDIM), jnp.float32),
            pltpu.VMEM((BLK, FDIM), jnp.float32),
            pltpu.SemaphoreType.DMA,
            pltpu.SemaphoreType.DMA,
        ],
    )(functools.partial(_lookup_body, nfull, ntail))
    return lookup(xt[0], xt[1], xt[2], tbl)


# speedup vs baseline: 5.6325x; 1.3500x over previous
import functools

import jax
import jax.numpy as jnp
from jax import lax
from jax.experimental import pallas as pl
from jax.experimental.pallas import tpu as pltpu
from jax.experimental.pallas import tpu_sc as plsc

FDIM = 32
G = 129
GG = G * G
V = G * G * G

NC = 2
NS = 16
NW = NC * NS
L = 16

BLK = 96

PPC = 8
NFULL_CHUNK = GG // PPC
CHUNK_VOX = PPC * G


VSTRIDE = FDIM + 1


def _relayout_body(fmv, out, inbuf, padbuf, outbuf, isem, osem):
    wid = lax.axis_index("s") * NC + lax.axis_index("c")
    nchunk = NFULL_CHUNK + 1
    kmax = -(-nchunk // NW)
    xi = lax.iota(jnp.int32, L)
    xstarts = [xg * L for xg in range(8)] + [G - L]
    bxv = [(xi + s) * VSTRIDE for s in xstarts]

    def do_plane(j, p0):
        jv = jnp.full((L,), j, jnp.int32)
        xv_tail = xi + (G - L)

        def c_body(c, carry2):
            for xg in range(8):
                v = inbuf[j, c, pl.ds(xstarts[xg], L)]
                plsc.store_scatter(padbuf, [bxv[xg] + c], v)
            cv = jnp.full((L,), c, jnp.int32)
            v = plsc.load_gather(inbuf, [jv, cv, xv_tail])
            plsc.store_scatter(padbuf, [bxv[8] + c], v)
            return carry2

        lax.fori_loop(0, FDIM, c_body, 0)
        joff = j * G * FDIM

        def x_body(x2, carry2):
            src = x2 * (2 * VSTRIDE)
            dst = joff + x2 * (2 * FDIM)
            outbuf[pl.ds(dst, L)] = padbuf[pl.ds(src, L)]
            outbuf[pl.ds(dst + L, L)] = padbuf[pl.ds(src + L, L)]
            outbuf[pl.ds(dst + 2 * L, L)] = padbuf[pl.ds(src + VSTRIDE, L)]
            outbuf[pl.ds(dst + 3 * L, L)] = \
                padbuf[pl.ds(src + VSTRIDE + L, L)]
            return carry2

        lax.fori_loop(0, (G - 1) // 2, x_body, 0)
        dst = joff + (G - 1) * FDIM
        src = (G - 1) * VSTRIDE
        outbuf[pl.ds(dst, L)] = padbuf[pl.ds(src, L)]
        outbuf[pl.ds(dst + L, L)] = padbuf[pl.ds(src + L, L)]
        copy = pltpu.make_async_copy(
            outbuf.at[pl.ds(joff, G * FDIM)],
            out.at[pl.ds((p0 + j) * G * FDIM, G * FDIM)], osem)
        copy.start()
        return copy

    def chunk_body(k, carry):
        t = k * NW + wid

        @pl.when(t < NFULL_CHUNK)
        def _full():
            p0 = t * PPC
            incopies = []
            for j in range(PPC):
                incopies.append(pltpu.make_async_copy(
                    fmv.at[p0 + j], inbuf.at[j], isem))
                incopies[-1].start()
            outcopies = []
            for j in range(PPC):
                incopies[j].wait()
                outcopies.append(do_plane(j, p0))
            for c in outcopies:
                c.wait()

        @pl.when(t == NFULL_CHUNK)
        def _tail():
            p0 = NFULL_CHUNK * PPC
            copy = pltpu.make_async_copy(fmv.at[p0], inbuf.at[0], isem)
            copy.start()
            copy.wait()
            do_plane(0, p0).wait()

        return carry

    lax.fori_loop(0, kmax, chunk_body, 0)


def _lerp_groups(cx_v, cy_v, cz_v, idx_v, w_v, ngroups):
    gmax = jnp.float32(G - 1)
    for g in range(ngroups):
        sl = pl.ds(g * L, L)
        ix = jnp.clip((cx_v[sl] + 1.0) * (0.5 * (G - 1)), 0.0, gmax)
        iy = jnp.clip((cy_v[sl] + 1.0) * (0.5 * (G - 1)), 0.0, gmax)
        iz = jnp.clip((cz_v[sl] + 1.0) * (0.5 * (G - 1)), 0.0, gmax)
        x0 = ix.astype(jnp.int32)
        y0 = iy.astype(jnp.int32)
        z0 = iz.astype(jnp.int32)
        fx = ix - x0.astype(jnp.float32)
        fy = iy - y0.astype(jnp.float32)
        fz = iz - z0.astype(jnp.float32)
        dx = jnp.minimum(x0 + 1, G - 1) - x0
        dy = (jnp.minimum(y0 + 1, G - 1) - y0) * G
        dz = (jnp.minimum(z0 + 1, G - 1) - z0) * GG
        base000 = z0 * GG + y0 * G + x0
        gx = (1.0 - fx, fx)
        gy = (1.0 - fy, fy)
        gz = (1.0 - fz, fz)
        ox = (jnp.zeros((L,), jnp.int32), dx)
        oy = (jnp.zeros((L,), jnp.int32), dy)
        oz = (jnp.zeros((L,), jnp.int32), dz)
        k = 0
        for kz in range(2):
            for ky in range(2):
                for kx in range(2):
                    idx_v[k, sl] = base000 + oz[kz] + oy[ky] + ox[kx]
                    w_v[k, sl] = gz[kz] * gy[ky] * gx[kx]
                    k += 1


def _combine_group(g, w_v, corners_v, out_v):
    sl = pl.ds(g * L, L)
    w_rows = [w_v[k, sl] for k in range(8)]
    for q in range(L):
        p = g * L + q
        acc0 = jnp.zeros((L,), jnp.float32)
        acc1 = jnp.zeros((L,), jnp.float32)
        for k in range(8):
            wk = w_rows[k][q]
            acc0 = acc0 + wk * corners_v[k, p, pl.ds(0, L)]
            acc1 = acc1 + wk * corners_v[k, p, pl.ds(L, L)]
        out_v[p, pl.ds(0, L)] = acc0
        out_v[p, pl.ds(L, L)] = acc1


def _lookup_body(nfull, ntail, xc, yc, zc, tbl, out, cx_v, cy_v, cz_v, idx_v,
                 w_v, corners_v, out_v, gsem0, gsem1, osem0, osem1):
    wid = lax.axis_index("s") * NC + lax.axis_index("c")
    gsems = (gsem0, gsem1)
    osems = (osem0, osem1)
    kmax = -(-nfull // NW)
    kmax2 = kmax // 2 + 1

    def valid(i):
        return i * NW + wid < nfull

    def prefetch(i, p):
        @pl.when(valid(i))
        def _():
            base = (i * NW + wid) * BLK
            pltpu.sync_copy(xc.at[pl.ds(base, BLK)], cx_v)
            pltpu.sync_copy(yc.at[pl.ds(base, BLK)], cy_v)
            pltpu.sync_copy(zc.at[pl.ds(base, BLK)], cz_v)
            _lerp_groups(cx_v, cy_v, cz_v, idx_v.at[p], w_v.at[p], BLK // L)
            for k in range(8):
                pltpu.async_copy(tbl.at[idx_v.at[p, k]],
                                 corners_v.at[p, k], gsems[p])

    def consume(i, p):
        @pl.when(valid(i))
        def _():
            for k in range(8):
                pltpu.make_async_copy(tbl.at[idx_v.at[p, 0]],
                                      corners_v.at[p, 0], gsems[p]).wait()

            def group_body(g, carry2):
                _combine_group(g, w_v.at[p], corners_v.at[p], out_v.at[p])
                return carry2

            lax.fori_loop(0, BLK // L, group_body, 0)
            base = (i * NW + wid) * BLK
            pltpu.make_async_copy(out_v.at[p], out.at[pl.ds(base, BLK)],
                                  osems[p]).start()

        @pl.when(jnp.logical_and(i >= 1, valid(i - 1)))
        def _retire():
            q = 1 - p
            base = ((i - 1) * NW + wid) * BLK
            pltpu.make_async_copy(out_v.at[q], out.at[pl.ds(base, BLK)],
                                  osems[q]).wait()

    prefetch(0, 0)

    def step(i2, carry):
        i = i2 * 2
        prefetch(i + 1, 1)
        consume(i, 0)
        prefetch(i + 2, 0)
        consume(i + 1, 1)
        return carry

    lax.fori_loop(0, kmax2, step, 0)

    if ntail:
        @pl.when(wid == 0)
        def _tail():
            base = nfull * BLK
            pltpu.sync_copy(xc.at[pl.ds(base, ntail)],
                            cx_v.at[pl.ds(0, ntail)])
            pltpu.sync_copy(yc.at[pl.ds(base, ntail)],
                            cy_v.at[pl.ds(0, ntail)])
            pltpu.sync_copy(zc.at[pl.ds(base, ntail)],
                            cz_v.at[pl.ds(0, ntail)])
            _lerp_groups(cx_v, cy_v, cz_v, idx_v.at[0], w_v.at[0],
                         ntail // L)
            for k in range(8):
                pltpu.async_copy(tbl.at[idx_v.at[0, k, pl.ds(0, ntail)]],
                                 corners_v.at[0, k, pl.ds(0, ntail), :],
                                 gsem0)
            for k in range(8):
                pltpu.make_async_copy(
                    tbl.at[idx_v.at[0, 0, pl.ds(0, ntail)]],
                    corners_v.at[0, 0, pl.ds(0, ntail), :], gsem0).wait()

            def group_body(g, carry2):
                _combine_group(g, w_v.at[0], corners_v.at[0], out_v.at[0])
                return carry2

            lax.fori_loop(0, ntail // L, group_body, 0)
            copy = pltpu.make_async_copy(
                out_v.at[0, pl.ds(0, ntail), :],
                out.at[pl.ds(base, ntail)], osem0)
            copy.start()
            copy.wait()


def kernel(x, fm):
    n = x.shape[0]
    nfull = n // BLK
    ntail = n - nfull * BLK

    mesh = plsc.VectorSubcoreMesh(core_axis_name="c", subcore_axis_name="s")

    fmv = jnp.transpose(fm[0], (1, 2, 0, 3)).reshape(GG, FDIM, G)
    relayout = functools.partial(
        pl.kernel,
        mesh=mesh,
        compiler_params=pltpu.CompilerParams(use_tc_tiling_on_sc=True,
                                             needs_layout_passes=False),
        out_type=jax.ShapeDtypeStruct((V * FDIM,), jnp.float32),
        scratch_types=[
            pltpu.VMEM((PPC, FDIM, G), jnp.float32),
            pltpu.VMEM((G * VSTRIDE,), jnp.float32),
            pltpu.VMEM((CHUNK_VOX * FDIM,), jnp.float32),
            pltpu.SemaphoreType.DMA,
            pltpu.SemaphoreType.DMA,
        ],
    )(_relayout_body)
    tbl = relayout(fmv).reshape(V, FDIM)

    xt = x.T
    lookup = functools.partial(
        pl.kernel,
        mesh=mesh,
        compiler_params=pltpu.CompilerParams(use_tc_tiling_on_sc=False,
                                             needs_layout_passes=False),
        out_type=jax.ShapeDtypeStruct((n, FDIM), jnp.float32),
        scratch_types=[
            pltpu.VMEM((BLK,), jnp.float32),
            pltpu.VMEM((BLK,), jnp.float32),
            pltpu.VMEM((BLK,), jnp.float32),
            pltpu.VMEM((2, 8, BLK), jnp.int32),
            pltpu.VMEM((2, 8, BLK), jnp.float32),
            pltpu.VMEM((2, 8, BLK, FDIM), jnp.float32),
            pltpu.VMEM((2, BLK, FDIM), jnp.float32),
            pltpu.SemaphoreType.DMA,
            pltpu.SemaphoreType.DMA,
            pltpu.SemaphoreType.DMA,
            pltpu.SemaphoreType.DMA,
        ],
    )(functools.partial(_lookup_body, nfull, ntail))
    return lookup(xt[0], xt[1], xt[2], tbl)


# speedup vs baseline: 5.8417x; 1.0371x over previous
import functools

import jax
import jax.numpy as jnp
from jax import lax
from jax.experimental import pallas as pl
from jax.experimental.pallas import tpu as pltpu
from jax.experimental.pallas import tpu_sc as plsc

FDIM = 32
G = 129
GG = G * G
V = G * G * G

NC = 2
NS = 16
NW = NC * NS
L = 16

BLK = 128

VSTRIDE = FDIM + 1

PPC = 8
NFULL_CHUNK = GG // PPC
CHUNK_VOX = PPC * G


def _relayout_body(fmv, out, inbuf, padbuf, outbuf, isem, osem):
    wid = lax.axis_index("s") * NC + lax.axis_index("c")
    nchunk = NFULL_CHUNK + 1
    kmax = -(-nchunk // NW)
    xi = lax.iota(jnp.int32, L)
    xstarts = [xg * L for xg in range(8)] + [G - L]
    bxv = [(xi + s) * VSTRIDE for s in xstarts]

    def do_plane(j, p0):
        jv = jnp.full((L,), j, jnp.int32)
        xv_tail = xi + (G - L)

        def c_body(c, carry2):
            for xg in range(8):
                v = inbuf[j, c, pl.ds(xstarts[xg], L)]
                plsc.store_scatter(padbuf, [bxv[xg] + c], v)
            cv = jnp.full((L,), c, jnp.int32)
            v = plsc.load_gather(inbuf, [jv, cv, xv_tail])
            plsc.store_scatter(padbuf, [bxv[8] + c], v)
            return carry2

        lax.fori_loop(0, FDIM, c_body, 0, unroll=2)
        joff = j * G * FDIM

        def x_body(x2, carry2):
            src = x2 * (2 * VSTRIDE)
            dst = joff + x2 * (2 * FDIM)
            outbuf[pl.ds(dst, L)] = padbuf[pl.ds(src, L)]
            outbuf[pl.ds(dst + L, L)] = padbuf[pl.ds(src + L, L)]
            outbuf[pl.ds(dst + 2 * L, L)] = padbuf[pl.ds(src + VSTRIDE, L)]
            outbuf[pl.ds(dst + 3 * L, L)] = \
                padbuf[pl.ds(src + VSTRIDE + L, L)]
            return carry2

        lax.fori_loop(0, (G - 1) // 2, x_body, 0, unroll=2)
        dst = joff + (G - 1) * FDIM
        src = (G - 1) * VSTRIDE
        outbuf[pl.ds(dst, L)] = padbuf[pl.ds(src, L)]
        outbuf[pl.ds(dst + L, L)] = padbuf[pl.ds(src + L, L)]
        copy = pltpu.make_async_copy(
            outbuf.at[pl.ds(joff, G * FDIM)],
            out.at[pl.ds((p0 + j) * G * FDIM, G * FDIM)], osem)
        copy.start()
        return copy

    def chunk_body(k, carry):
        t = k * NW + wid

        @pl.when(t < NFULL_CHUNK)
        def _full():
            p0 = t * PPC
            incopies = []
            for j in range(PPC):
                incopies.append(pltpu.make_async_copy(
                    fmv.at[p0 + j], inbuf.at[j], isem))
                incopies[-1].start()
            outcopies = []
            for j in range(PPC):
                incopies[j].wait()
                outcopies.append(do_plane(j, p0))
            for c in outcopies:
                c.wait()

        @pl.when(t == NFULL_CHUNK)
        def _tail():
            p0 = NFULL_CHUNK * PPC
            copy = pltpu.make_async_copy(fmv.at[p0], inbuf.at[0], isem)
            copy.start()
            copy.wait()
            do_plane(0, p0).wait()

        return carry

    lax.fori_loop(0, kmax, chunk_body, 0)


def _lerp_groups(cx_v, cy_v, cz_v, idx_v, w_v, ngroups):
    gmax = jnp.float32(G - 1)
    for g in range(ngroups):
        sl = pl.ds(g * L, L)
        ix = jnp.clip((cx_v[sl] + 1.0) * (0.5 * (G - 1)), 0.0, gmax)
        iy = jnp.clip((cy_v[sl] + 1.0) * (0.5 * (G - 1)), 0.0, gmax)
        iz = jnp.clip((cz_v[sl] + 1.0) * (0.5 * (G - 1)), 0.0, gmax)
        x0 = ix.astype(jnp.int32)
        y0 = iy.astype(jnp.int32)
        z0 = iz.astype(jnp.int32)
        fx = ix - x0.astype(jnp.float32)
        fy = iy - y0.astype(jnp.float32)
        fz = iz - z0.astype(jnp.float32)
        dx = jnp.minimum(x0 + 1, G - 1) - x0
        dy = (jnp.minimum(y0 + 1, G - 1) - y0) * G
        dz = (jnp.minimum(z0 + 1, G - 1) - z0) * GG
        base000 = z0 * GG + y0 * G + x0
        gx = (1.0 - fx, fx)
        gy = (1.0 - fy, fy)
        gz = (1.0 - fz, fz)
        ox = (jnp.zeros((L,), jnp.int32), dx)
        oy = (jnp.zeros((L,), jnp.int32), dy)
        oz = (jnp.zeros((L,), jnp.int32), dz)
        k = 0
        for kz in range(2):
            for ky in range(2):
                for kx in range(2):
                    idx_v[k, sl] = base000 + oz[kz] + oy[ky] + ox[kx]
                    w_v[k, sl] = gz[kz] * gy[ky] * gx[kx]
                    k += 1


def _combine_group(g, w_v, corners_v, out_v):
    sl = pl.ds(g * L, L)
    w_rows = [w_v[k, sl] for k in range(8)]
    for q in range(L):
        p = g * L + q
        acc0 = jnp.zeros((L,), jnp.float32)
        acc1 = jnp.zeros((L,), jnp.float32)
        for k in range(8):
            wk = w_rows[k][q]
            acc0 = acc0 + wk * corners_v[k, p, pl.ds(0, L)]
            acc1 = acc1 + wk * corners_v[k, p, pl.ds(L, L)]
        out_v[p, pl.ds(0, L)] = acc0
        out_v[p, pl.ds(L, L)] = acc1


def _lookup_body(nfull, ntail, xc, yc, zc, tbl, out, cx_v, cy_v, cz_v, idx_v,
                 w_v, corners_v, out_v, gsem0, gsem1, osem0, osem1):
    wid = lax.axis_index("s") * NC + lax.axis_index("c")
    gsems = (gsem0, gsem1)
    osems = (osem0, osem1)
    kmax = -(-nfull // NW)
    kmax2 = kmax // 2 + 1

    def valid(i):
        return i * NW + wid < nfull

    def prefetch(i, p):
        @pl.when(valid(i))
        def _():
            base = (i * NW + wid) * BLK
            pltpu.sync_copy(xc.at[pl.ds(base, BLK)], cx_v)
            pltpu.sync_copy(yc.at[pl.ds(base, BLK)], cy_v)
            pltpu.sync_copy(zc.at[pl.ds(base, BLK)], cz_v)
            _lerp_groups(cx_v, cy_v, cz_v, idx_v.at[p], w_v.at[p], BLK // L)
            for k in range(8):
                pltpu.async_copy(tbl.at[idx_v.at[p, k]],
                                 corners_v.at[p, k], gsems[p])

    def consume(i, p):
        @pl.when(valid(i))
        def _():
            for k in range(8):
                pltpu.make_async_copy(tbl.at[idx_v.at[p, 0]],
                                      corners_v.at[p, 0], gsems[p]).wait()

            def group_body(g, carry2):
                _combine_group(g, w_v.at[p], corners_v.at[p], out_v.at[p])
                return carry2

            lax.fori_loop(0, BLK // L, group_body, 0)
            base = (i * NW + wid) * BLK
            pltpu.make_async_copy(out_v.at[p], out.at[pl.ds(base, BLK)],
                                  osems[p]).start()

        @pl.when(jnp.logical_and(i >= 1, valid(i - 1)))
        def _retire():
            q = 1 - p
            base = ((i - 1) * NW + wid) * BLK
            pltpu.make_async_copy(out_v.at[q], out.at[pl.ds(base, BLK)],
                                  osems[q]).wait()

    prefetch(0, 0)

    def step(i2, carry):
        i = i2 * 2
        prefetch(i + 1, 1)
        consume(i, 0)
        prefetch(i + 2, 0)
        consume(i + 1, 1)
        return carry

    lax.fori_loop(0, kmax2, step, 0)

    if ntail:
        @pl.when(wid == 0)
        def _tail():
            base = nfull * BLK
            pltpu.sync_copy(xc.at[pl.ds(base, ntail)],
                            cx_v.at[pl.ds(0, ntail)])
            pltpu.sync_copy(yc.at[pl.ds(base, ntail)],
                            cy_v.at[pl.ds(0, ntail)])
            pltpu.sync_copy(zc.at[pl.ds(base, ntail)],
                            cz_v.at[pl.ds(0, ntail)])
            _lerp_groups(cx_v, cy_v, cz_v, idx_v.at[0], w_v.at[0],
                         ntail // L)
            for k in range(8):
                pltpu.async_copy(tbl.at[idx_v.at[0, k, pl.ds(0, ntail)]],
                                 corners_v.at[0, k, pl.ds(0, ntail), :],
                                 gsem0)
            for k in range(8):
                pltpu.make_async_copy(
                    tbl.at[idx_v.at[0, 0, pl.ds(0, ntail)]],
                    corners_v.at[0, 0, pl.ds(0, ntail), :], gsem0).wait()

            def group_body(g, carry2):
                _combine_group(g, w_v.at[0], corners_v.at[0], out_v.at[0])
                return carry2

            lax.fori_loop(0, ntail // L, group_body, 0)
            copy = pltpu.make_async_copy(
                out_v.at[0, pl.ds(0, ntail), :],
                out.at[pl.ds(base, ntail)], osem0)
            copy.start()
            copy.wait()


def kernel(x, fm):
    n = x.shape[0]
    nfull = n // BLK
    ntail = n - nfull * BLK

    mesh = plsc.VectorSubcoreMesh(core_axis_name="c", subcore_axis_name="s")

    fmv = jnp.transpose(fm[0], (1, 2, 0, 3)).reshape(GG, FDIM, G)
    relayout = functools.partial(
        pl.kernel,
        mesh=mesh,
        compiler_params=pltpu.CompilerParams(use_tc_tiling_on_sc=True,
                                             needs_layout_passes=False),
        out_type=jax.ShapeDtypeStruct((V * FDIM,), jnp.float32),
        scratch_types=[
            pltpu.VMEM((PPC, FDIM, G), jnp.float32),
            pltpu.VMEM((G * VSTRIDE,), jnp.float32),
            pltpu.VMEM((CHUNK_VOX * FDIM,), jnp.float32),
            pltpu.SemaphoreType.DMA,
            pltpu.SemaphoreType.DMA,
        ],
    )(_relayout_body)
    tbl = relayout(fmv).reshape(V, FDIM)

    xt = x.T
    lookup = functools.partial(
        pl.kernel,
        mesh=mesh,
        compiler_params=pltpu.CompilerParams(use_tc_tiling_on_sc=False,
                                             needs_layout_passes=False),
        out_type=jax.ShapeDtypeStruct((n, FDIM), jnp.float32),
        scratch_types=[
            pltpu.VMEM((BLK,), jnp.float32),
            pltpu.VMEM((BLK,), jnp.float32),
            pltpu.VMEM((BLK,), jnp.float32),
            pltpu.VMEM((2, 8, BLK), jnp.int32),
            pltpu.VMEM((2, 8, BLK), jnp.float32),
            pltpu.VMEM((2, 8, BLK, FDIM), jnp.float32),
            pltpu.VMEM((2, BLK, FDIM), jnp.float32),
            pltpu.SemaphoreType.DMA,
            pltpu.SemaphoreType.DMA,
            pltpu.SemaphoreType.DMA,
            pltpu.SemaphoreType.DMA,
        ],
    )(functools.partial(_lookup_body, nfull, ntail))
    return lookup(xt[0], xt[1], xt[2], tbl)


# speedup vs baseline: 6.0429x; 1.0345x over previous
import functools

import jax
import jax.numpy as jnp
from jax import lax
from jax.experimental import pallas as pl
from jax.experimental.pallas import tpu as pltpu
from jax.experimental.pallas import tpu_sc as plsc

FDIM = 32
G = 129
GG = G * G
V = G * G * G

NC = 2
NS = 16
NW = NC * NS
L = 16

BLK = 128

VSTRIDE = FDIM + 1

PPC = 4
NFULL_CHUNK = GG // PPC
CHUNK_VOX = PPC * G


def _relayout_body(fmv, out, inbuf, padbuf, outbuf, isem0, isem1, osem0,
                   osem1):
    wid = lax.axis_index("s") * NC + lax.axis_index("c")
    isems = (isem0, isem1)
    osems = (osem0, osem1)
    nchunk = NFULL_CHUNK + 1
    kmax2 = -(-nchunk // NW) // 2 + 1
    xi = lax.iota(jnp.int32, L)
    xstarts = [xg * L for xg in range(8)] + [G - L]
    bxv = [(xi + s) * VSTRIDE for s in xstarts]

    def do_plane(b, j, p0, osem):
        jv = jnp.full((L,), j, jnp.int32)
        xv_tail = xi + (G - L)

        def c_body(c, carry2):
            for xg in range(8):
                v = inbuf[b, j, c, pl.ds(xstarts[xg], L)]
                plsc.store_scatter(padbuf, [bxv[xg] + c], v)
            cv = jnp.full((L,), c, jnp.int32)
            v = plsc.load_gather(inbuf.at[b], [jv, cv, xv_tail])
            plsc.store_scatter(padbuf, [bxv[8] + c], v)
            return carry2

        lax.fori_loop(0, FDIM, c_body, 0, unroll=4)
        joff = (b * PPC + j) * G * FDIM

        def x_body(x2, carry2):
            src = x2 * (2 * VSTRIDE)
            dst = joff + x2 * (2 * FDIM)
            outbuf[pl.ds(dst, L)] = padbuf[pl.ds(src, L)]
            outbuf[pl.ds(dst + L, L)] = padbuf[pl.ds(src + L, L)]
            outbuf[pl.ds(dst + 2 * L, L)] = \
                padbuf[pl.ds(src + VSTRIDE, L)]
            outbuf[pl.ds(dst + 3 * L, L)] = \
                padbuf[pl.ds(src + VSTRIDE + L, L)]
            return carry2

        lax.fori_loop(0, (G - 1) // 2, x_body, 0, unroll=4)
        dst = joff + (G - 1) * FDIM
        src = (G - 1) * VSTRIDE
        outbuf[pl.ds(dst, L)] = padbuf[pl.ds(src, L)]
        outbuf[pl.ds(dst + L, L)] = padbuf[pl.ds(src + L, L)]
        pltpu.make_async_copy(
            outbuf.at[pl.ds(joff, G * FDIM)],
            out.at[pl.ds((p0 + j) * G * FDIM, G * FDIM)], osem).start()

    def in_descr(b, j, t):
        return pltpu.make_async_copy(fmv.at[t * PPC + j], inbuf.at[b, j],
                                     isems[b])

    def out_descr(b, j, t):
        return pltpu.make_async_copy(
            outbuf.at[pl.ds((b * PPC + j) * G * FDIM, G * FDIM)],
            out.at[pl.ds((t * PPC + j) * G * FDIM, G * FDIM)], osems[b])

    def prefetch(k, b):
        t = k * NW + wid

        @pl.when(t < NFULL_CHUNK)
        def _full():
            for j in range(PPC):
                in_descr(b, j, t).start()

        @pl.when(t == NFULL_CHUNK)
        def _tail():
            in_descr(b, 0, t).start()

    def consume(k, b):
        t = k * NW + wid

        @pl.when(t < NFULL_CHUNK)
        def _full():
            for j in range(PPC):
                in_descr(b, j, t).wait()
                do_plane(b, j, t * PPC, osems[b])

        @pl.when(t == NFULL_CHUNK)
        def _tail():
            in_descr(b, 0, t).wait()
            do_plane(b, 0, t * PPC, osems[b])

        tp = (k - 1) * NW + wid
        q = 1 - b

        @pl.when(jnp.logical_and(k >= 1, tp < NFULL_CHUNK))
        def _retire_full():
            for j in range(PPC):
                out_descr(q, j, tp).wait()

        @pl.when(jnp.logical_and(k >= 1, tp == NFULL_CHUNK))
        def _retire_tail():
            out_descr(q, 0, tp).wait()

    prefetch(0, 0)

    def step(k2, carry):
        k = k2 * 2
        prefetch(k + 1, 1)
        consume(k, 0)
        prefetch(k + 2, 0)
        consume(k + 1, 1)
        return carry

    lax.fori_loop(0, kmax2, step, 0)


def _lerp_groups(cx_v, cy_v, cz_v, idx_v, w_v, ngroups):
    gmax = jnp.float32(G - 1)
    for g in range(ngroups):
        sl = pl.ds(g * L, L)
        ix = jnp.clip((cx_v[sl] + 1.0) * (0.5 * (G - 1)), 0.0, gmax)
        iy = jnp.clip((cy_v[sl] + 1.0) * (0.5 * (G - 1)), 0.0, gmax)
        iz = jnp.clip((cz_v[sl] + 1.0) * (0.5 * (G - 1)), 0.0, gmax)
        x0 = ix.astype(jnp.int32)
        y0 = iy.astype(jnp.int32)
        z0 = iz.astype(jnp.int32)
        fx = ix - x0.astype(jnp.float32)
        fy = iy - y0.astype(jnp.float32)
        fz = iz - z0.astype(jnp.float32)
        dx = jnp.minimum(x0 + 1, G - 1) - x0
        dy = (jnp.minimum(y0 + 1, G - 1) - y0) * G
        dz = (jnp.minimum(z0 + 1, G - 1) - z0) * GG
        base000 = z0 * GG + y0 * G + x0
        gx = (1.0 - fx, fx)
        gy = (1.0 - fy, fy)
        gz = (1.0 - fz, fz)
        ox = (jnp.zeros((L,), jnp.int32), dx)
        oy = (jnp.zeros((L,), jnp.int32), dy)
        oz = (jnp.zeros((L,), jnp.int32), dz)
        k = 0
        for kz in range(2):
            for ky in range(2):
                for kx in range(2):
                    idx_v[k, sl] = base000 + oz[kz] + oy[ky] + ox[kx]
                    w_v[k, sl] = gz[kz] * gy[ky] * gx[kx]
                    k += 1


def _combine_group(g, w_v, corners_v, out_v):
    sl = pl.ds(g * L, L)
    w_rows = [w_v[k, sl] for k in range(8)]
    for q in range(L):
        p = g * L + q
        acc0 = jnp.zeros((L,), jnp.float32)
        acc1 = jnp.zeros((L,), jnp.float32)
        for k in range(8):
            wk = w_rows[k][q]
            acc0 = acc0 + wk * corners_v[k, p, pl.ds(0, L)]
            acc1 = acc1 + wk * corners_v[k, p, pl.ds(L, L)]
        out_v[p, pl.ds(0, L)] = acc0
        out_v[p, pl.ds(L, L)] = acc1


def _lookup_body(nfull, ntail, xc, yc, zc, tbl, out, cx_v, cy_v, cz_v, idx_v,
                 w_v, corners_v, out_v, gsem0, gsem1, osem0, osem1):
    wid = lax.axis_index("s") * NC + lax.axis_index("c")
    gsems = (gsem0, gsem1)
    osems = (osem0, osem1)
    kmax = -(-nfull // NW)
    kmax2 = kmax // 2 + 1

    def valid(i):
        return i * NW + wid < nfull

    def prefetch(i, p):
        @pl.when(valid(i))
        def _():
            base = (i * NW + wid) * BLK
            pltpu.sync_copy(xc.at[pl.ds(base, BLK)], cx_v)
            pltpu.sync_copy(yc.at[pl.ds(base, BLK)], cy_v)
            pltpu.sync_copy(zc.at[pl.ds(base, BLK)], cz_v)
            _lerp_groups(cx_v, cy_v, cz_v, idx_v.at[p], w_v.at[p], BLK // L)
            for k in range(8):
                pltpu.async_copy(tbl.at[idx_v.at[p, k]],
                                 corners_v.at[p, k], gsems[p])

    def consume(i, p):
        @pl.when(valid(i))
        def _():
            for k in range(8):
                pltpu.make_async_copy(tbl.at[idx_v.at[p, 0]],
                                      corners_v.at[p, 0], gsems[p]).wait()

            def group_body(g, carry2):
                _combine_group(g, w_v.at[p], corners_v.at[p], out_v.at[p])
                return carry2

            lax.fori_loop(0, BLK // L, group_body, 0)
            base = (i * NW + wid) * BLK
            pltpu.make_async_copy(out_v.at[p], out.at[pl.ds(base, BLK)],
                                  osems[p]).start()

        @pl.when(jnp.logical_and(i >= 1, valid(i - 1)))
        def _retire():
            q = 1 - p
            base = ((i - 1) * NW + wid) * BLK
            pltpu.make_async_copy(out_v.at[q], out.at[pl.ds(base, BLK)],
                                  osems[q]).wait()

    prefetch(0, 0)

    def step(i2, carry):
        i = i2 * 2
        prefetch(i + 1, 1)
        consume(i, 0)
        prefetch(i + 2, 0)
        consume(i + 1, 1)
        return carry

    lax.fori_loop(0, kmax2, step, 0)

    if ntail:
        @pl.when(wid == 0)
        def _tail():
            base = nfull * BLK
            pltpu.sync_copy(xc.at[pl.ds(base, ntail)],
                            cx_v.at[pl.ds(0, ntail)])
            pltpu.sync_copy(yc.at[pl.ds(base, ntail)],
                            cy_v.at[pl.ds(0, ntail)])
            pltpu.sync_copy(zc.at[pl.ds(base, ntail)],
                            cz_v.at[pl.ds(0, ntail)])
            _lerp_groups(cx_v, cy_v, cz_v, idx_v.at[0], w_v.at[0],
                         ntail // L)
            for k in range(8):
                pltpu.async_copy(tbl.at[idx_v.at[0, k, pl.ds(0, ntail)]],
                                 corners_v.at[0, k, pl.ds(0, ntail), :],
                                 gsem0)
            for k in range(8):
                pltpu.make_async_copy(
                    tbl.at[idx_v.at[0, 0, pl.ds(0, ntail)]],
                    corners_v.at[0, 0, pl.ds(0, ntail), :], gsem0).wait()

            def group_body(g, carry2):
                _combine_group(g, w_v.at[0], corners_v.at[0], out_v.at[0])
                return carry2

            lax.fori_loop(0, ntail // L, group_body, 0)
            copy = pltpu.make_async_copy(
                out_v.at[0, pl.ds(0, ntail), :],
                out.at[pl.ds(base, ntail)], osem0)
            copy.start()
            copy.wait()


def kernel(x, fm):
    n = x.shape[0]
    nfull = n // BLK
    ntail = n - nfull * BLK

    mesh = plsc.VectorSubcoreMesh(core_axis_name="c", subcore_axis_name="s")

    fmv = jnp.transpose(fm[0], (1, 2, 0, 3)).reshape(GG, FDIM, G)
    relayout = functools.partial(
        pl.kernel,
        mesh=mesh,
        compiler_params=pltpu.CompilerParams(use_tc_tiling_on_sc=True,
                                             needs_layout_passes=False),
        out_type=jax.ShapeDtypeStruct((V * FDIM,), jnp.float32),
        scratch_types=[
            pltpu.VMEM((2, PPC, FDIM, G), jnp.float32),
            pltpu.VMEM((G * VSTRIDE,), jnp.float32),
            pltpu.VMEM((2 * CHUNK_VOX * FDIM,), jnp.float32),
            pltpu.SemaphoreType.DMA,
            pltpu.SemaphoreType.DMA,
            pltpu.SemaphoreType.DMA,
            pltpu.SemaphoreType.DMA,
        ],
    )(_relayout_body)
    tbl = relayout(fmv).reshape(V, FDIM)

    xt = x.T
    lookup = functools.partial(
        pl.kernel,
        mesh=mesh,
        compiler_params=pltpu.CompilerParams(use_tc_tiling_on_sc=False,
                                             needs_layout_passes=False),
        out_type=jax.ShapeDtypeStruct((n, FDIM), jnp.float32),
        scratch_types=[
            pltpu.VMEM((BLK,), jnp.float32),
            pltpu.VMEM((BLK,), jnp.float32),
            pltpu.VMEM((BLK,), jnp.float32),
            pltpu.VMEM((2, 8, BLK), jnp.int32),
            pltpu.VMEM((2, 8, BLK), jnp.float32),
            pltpu.VMEM((2, 8, BLK, FDIM), jnp.float32),
            pltpu.VMEM((2, BLK, FDIM), jnp.float32),
            pltpu.SemaphoreType.DMA,
            pltpu.SemaphoreType.DMA,
            pltpu.SemaphoreType.DMA,
            pltpu.SemaphoreType.DMA,
        ],
    )(functools.partial(_lookup_body, nfull, ntail))
    return lookup(xt[0], xt[1], xt[2], tbl)


# speedup vs baseline: 6.2842x; 1.0399x over previous
import functools

import jax
import jax.numpy as jnp
from jax import lax
from jax.experimental import pallas as pl
from jax.experimental.pallas import tpu as pltpu
from jax.experimental.pallas import tpu_sc as plsc

FDIM = 32
G = 129
GG = G * G
V = G * G * G

NC = 2
NS = 16
NW = NC * NS
L = 16

BLK = 128

VSTRIDE = FDIM + 1

PPC = 4
NFULL_CHUNK = GG // PPC
CHUNK_VOX = PPC * G


def _relayout_body(fmv, out, inbuf, padbuf, outbuf, isem0, isem1, osem0,
                   osem1):
    wid = lax.axis_index("s") * NC + lax.axis_index("c")
    isems = (isem0, isem1)
    osems = (osem0, osem1)
    nchunk = NFULL_CHUNK + 1
    kmax2 = -(-nchunk // NW) // 2 + 1
    xi = lax.iota(jnp.int32, L)
    xstarts = [xg * L for xg in range(8)] + [G - L]
    bxv = [(xi + s) * VSTRIDE for s in xstarts]

    def do_plane(b, j, p0, osem):
        jv = jnp.full((L,), j, jnp.int32)
        xv_tail = xi + (G - L)

        def c_body(c, carry2):
            for xg in range(8):
                v = inbuf[b, j, c, pl.ds(xstarts[xg], L)]
                plsc.store_scatter(padbuf, [bxv[xg] + c], v)
            cv = jnp.full((L,), c, jnp.int32)
            v = plsc.load_gather(inbuf.at[b], [jv, cv, xv_tail])
            plsc.store_scatter(padbuf, [bxv[8] + c], v)
            return carry2

        lax.fori_loop(0, FDIM, c_body, 0, unroll=4)
        joff = (b * PPC + j) * G * FDIM

        def x_body(x2, carry2):
            src = x2 * (2 * VSTRIDE)
            dst = joff + x2 * (2 * FDIM)
            outbuf[pl.ds(dst, L)] = padbuf[pl.ds(src, L)]
            outbuf[pl.ds(dst + L, L)] = padbuf[pl.ds(src + L, L)]
            outbuf[pl.ds(dst + 2 * L, L)] = \
                padbuf[pl.ds(src + VSTRIDE, L)]
            outbuf[pl.ds(dst + 3 * L, L)] = \
                padbuf[pl.ds(src + VSTRIDE + L, L)]
            return carry2

        lax.fori_loop(0, (G - 1) // 2, x_body, 0, unroll=4)
        dst = joff + (G - 1) * FDIM
        src = (G - 1) * VSTRIDE
        outbuf[pl.ds(dst, L)] = padbuf[pl.ds(src, L)]
        outbuf[pl.ds(dst + L, L)] = padbuf[pl.ds(src + L, L)]
        pltpu.make_async_copy(
            outbuf.at[pl.ds(joff, G * FDIM)],
            out.at[pl.ds((p0 + j) * G * FDIM, G * FDIM)], osem).start()

    def in_descr(b, j, t):
        return pltpu.make_async_copy(fmv.at[t * PPC + j], inbuf.at[b, j],
                                     isems[b])

    def out_descr(b, j, t):
        return pltpu.make_async_copy(
            outbuf.at[pl.ds((b * PPC + j) * G * FDIM, G * FDIM)],
            out.at[pl.ds((t * PPC + j) * G * FDIM, G * FDIM)], osems[b])

    def prefetch(k, b):
        t = k * NW + wid

        @pl.when(t < NFULL_CHUNK)
        def _full():
            for j in range(PPC):
                in_descr(b, j, t).start()

        @pl.when(t == NFULL_CHUNK)
        def _tail():
            in_descr(b, 0, t).start()

    def consume(k, b):
        t = k * NW + wid

        @pl.when(t < NFULL_CHUNK)
        def _full():
            for j in range(PPC):
                in_descr(b, j, t).wait()
                do_plane(b, j, t * PPC, osems[b])

        @pl.when(t == NFULL_CHUNK)
        def _tail():
            in_descr(b, 0, t).wait()
            do_plane(b, 0, t * PPC, osems[b])

        tp = (k - 1) * NW + wid
        q = 1 - b

        @pl.when(jnp.logical_and(k >= 1, tp < NFULL_CHUNK))
        def _retire_full():
            for j in range(PPC):
                out_descr(q, j, tp).wait()

        @pl.when(jnp.logical_and(k >= 1, tp == NFULL_CHUNK))
        def _retire_tail():
            out_descr(q, 0, tp).wait()

    prefetch(0, 0)

    def step(k2, carry):
        k = k2 * 2
        prefetch(k + 1, 1)
        consume(k, 0)
        prefetch(k + 2, 0)
        consume(k + 1, 1)
        return carry

    lax.fori_loop(0, kmax2, step, 0)


def _lerp_groups(cx_v, cy_v, cz_v, idx_v, w_v, ngroups):
    gmax = jnp.float32(G - 1)
    for g in range(ngroups):
        sl = pl.ds(g * L, L)
        ix = jnp.clip((cx_v[sl] + 1.0) * (0.5 * (G - 1)), 0.0, gmax)
        iy = jnp.clip((cy_v[sl] + 1.0) * (0.5 * (G - 1)), 0.0, gmax)
        iz = jnp.clip((cz_v[sl] + 1.0) * (0.5 * (G - 1)), 0.0, gmax)
        x0 = ix.astype(jnp.int32)
        y0 = iy.astype(jnp.int32)
        z0 = iz.astype(jnp.int32)
        fx = ix - x0.astype(jnp.float32)
        fy = iy - y0.astype(jnp.float32)
        fz = iz - z0.astype(jnp.float32)
        dx = jnp.minimum(x0 + 1, G - 1) - x0
        dy = (jnp.minimum(y0 + 1, G - 1) - y0) * G
        dz = (jnp.minimum(z0 + 1, G - 1) - z0) * GG
        base000 = z0 * GG + y0 * G + x0
        gx = (1.0 - fx, fx)
        gy = (1.0 - fy, fy)
        gz = (1.0 - fz, fz)
        ox = (jnp.zeros((L,), jnp.int32), dx)
        oy = (jnp.zeros((L,), jnp.int32), dy)
        oz = (jnp.zeros((L,), jnp.int32), dz)
        k = 0
        for kz in range(2):
            for ky in range(2):
                for kx in range(2):
                    idx_v[k, sl] = base000 + oz[kz] + oy[ky] + ox[kx]
                    w_v[k, sl] = gz[kz] * gy[ky] * gx[kx]
                    k += 1


def _combine_group(g, w_v, corners_v, out_v):
    sl = pl.ds(g * L, L)
    w_rows = [w_v[k, sl] for k in range(8)]
    for q in range(L):
        p = g * L + q
        acc0 = jnp.zeros((L,), jnp.float32)
        acc1 = jnp.zeros((L,), jnp.float32)
        for k in range(8):
            wk = w_rows[k][q]
            acc0 = acc0 + wk * corners_v[k, p, pl.ds(0, L)]
            acc1 = acc1 + wk * corners_v[k, p, pl.ds(L, L)]
        out_v[p, pl.ds(0, L)] = acc0
        out_v[p, pl.ds(L, L)] = acc1


def _lookup_body(nfull, ntail, xc, yc, zc, tbl, out, cx_v, cy_v, cz_v, idx_v,
                 w_v, corners_v, out_v, gsem0, gsem1, osem0, osem1):
    wid = lax.axis_index("s") * NC + lax.axis_index("c")
    gsems = (gsem0, gsem1)
    osems = (osem0, osem1)
    kmax = -(-nfull // NW)
    kmax2 = kmax // 2 + 1

    def valid(i):
        return i * NW + wid < nfull

    def prefetch(i, p):
        @pl.when(valid(i))
        def _():
            base = (i * NW + wid) * BLK
            ccopies = [
                pltpu.make_async_copy(xc.at[pl.ds(base, BLK)], cx_v,
                                      gsems[p]),
                pltpu.make_async_copy(yc.at[pl.ds(base, BLK)], cy_v,
                                      gsems[p]),
                pltpu.make_async_copy(zc.at[pl.ds(base, BLK)], cz_v,
                                      gsems[p]),
            ]
            for cc in ccopies:
                cc.start()
            for cc in ccopies:
                cc.wait()
            _lerp_groups(cx_v, cy_v, cz_v, idx_v.at[p], w_v.at[p], BLK // L)
            for k in range(8):
                pltpu.async_copy(tbl.at[idx_v.at[p, k]],
                                 corners_v.at[p, k], gsems[p])

    def consume(i, p):
        @pl.when(valid(i))
        def _():
            for k in range(8):
                pltpu.make_async_copy(tbl.at[idx_v.at[p, 0]],
                                      corners_v.at[p, 0], gsems[p]).wait()

            def group_body(g, carry2):
                _combine_group(g, w_v.at[p], corners_v.at[p], out_v.at[p])
                return carry2

            lax.fori_loop(0, BLK // L, group_body, 0, unroll=2)
            base = (i * NW + wid) * BLK
            pltpu.make_async_copy(out_v.at[p], out.at[pl.ds(base, BLK)],
                                  osems[p]).start()

        @pl.when(jnp.logical_and(i >= 1, valid(i - 1)))
        def _retire():
            q = 1 - p
            base = ((i - 1) * NW + wid) * BLK
            pltpu.make_async_copy(out_v.at[q], out.at[pl.ds(base, BLK)],
                                  osems[q]).wait()

    prefetch(0, 0)

    def step(i2, carry):
        i = i2 * 2
        prefetch(i + 1, 1)
        consume(i, 0)
        prefetch(i + 2, 0)
        consume(i + 1, 1)
        return carry

    lax.fori_loop(0, kmax2, step, 0)

    if ntail:
        @pl.when(wid == 0)
        def _tail():
            base = nfull * BLK
            pltpu.sync_copy(xc.at[pl.ds(base, ntail)],
                            cx_v.at[pl.ds(0, ntail)])
            pltpu.sync_copy(yc.at[pl.ds(base, ntail)],
                            cy_v.at[pl.ds(0, ntail)])
            pltpu.sync_copy(zc.at[pl.ds(base, ntail)],
                            cz_v.at[pl.ds(0, ntail)])
            _lerp_groups(cx_v, cy_v, cz_v, idx_v.at[0], w_v.at[0],
                         ntail // L)
            for k in range(8):
                pltpu.async_copy(tbl.at[idx_v.at[0, k, pl.ds(0, ntail)]],
                                 corners_v.at[0, k, pl.ds(0, ntail), :],
                                 gsem0)
            for k in range(8):
                pltpu.make_async_copy(
                    tbl.at[idx_v.at[0, 0, pl.ds(0, ntail)]],
                    corners_v.at[0, 0, pl.ds(0, ntail), :], gsem0).wait()

            def group_body(g, carry2):
                _combine_group(g, w_v.at[0], corners_v.at[0], out_v.at[0])
                return carry2

            lax.fori_loop(0, ntail // L, group_body, 0)
            copy = pltpu.make_async_copy(
                out_v.at[0, pl.ds(0, ntail), :],
                out.at[pl.ds(base, ntail)], osem0)
            copy.start()
            copy.wait()


def kernel(x, fm):
    n = x.shape[0]
    nfull = n // BLK
    ntail = n - nfull * BLK

    mesh = plsc.VectorSubcoreMesh(core_axis_name="c", subcore_axis_name="s")

    fmv = jnp.transpose(fm[0], (1, 2, 0, 3)).reshape(GG, FDIM, G)
    relayout = functools.partial(
        pl.kernel,
        mesh=mesh,
        compiler_params=pltpu.CompilerParams(use_tc_tiling_on_sc=True,
                                             needs_layout_passes=False),
        out_type=jax.ShapeDtypeStruct((V * FDIM,), jnp.float32),
        scratch_types=[
            pltpu.VMEM((2, PPC, FDIM, G), jnp.float32),
            pltpu.VMEM((G * VSTRIDE,), jnp.float32),
            pltpu.VMEM((2 * CHUNK_VOX * FDIM,), jnp.float32),
            pltpu.SemaphoreType.DMA,
            pltpu.SemaphoreType.DMA,
            pltpu.SemaphoreType.DMA,
            pltpu.SemaphoreType.DMA,
        ],
    )(_relayout_body)
    tbl = relayout(fmv).reshape(V, FDIM)

    xt = x.T
    lookup = functools.partial(
        pl.kernel,
        mesh=mesh,
        compiler_params=pltpu.CompilerParams(use_tc_tiling_on_sc=False,
                                             needs_layout_passes=False),
        out_type=jax.ShapeDtypeStruct((n, FDIM), jnp.float32),
        scratch_types=[
            pltpu.VMEM((BLK,), jnp.float32),
            pltpu.VMEM((BLK,), jnp.float32),
            pltpu.VMEM((BLK,), jnp.float32),
            pltpu.VMEM((2, 8, BLK), jnp.int32),
            pltpu.VMEM((2, 8, BLK), jnp.float32),
            pltpu.VMEM((2, 8, BLK, FDIM), jnp.float32),
            pltpu.VMEM((2, BLK, FDIM), jnp.float32),
            pltpu.SemaphoreType.DMA,
            pltpu.SemaphoreType.DMA,
            pltpu.SemaphoreType.DMA,
            pltpu.SemaphoreType.DMA,
        ],
    )(functools.partial(_lookup_body, nfull, ntail))
    return lookup(xt[0], xt[1], xt[2], tbl)
